# Initial kernel scaffold; baseline (speedup 1.0000x reference)
import jax
import jax.numpy as jnp
from jax.experimental import pallas as pl


def kernel(video_clips):
    raise NotImplementedError("write your pallas kernel here")


# speedup vs baseline: 1.6863x; 1.6863x over previous
import numpy as np
import jax
import jax.numpy as jnp
from jax.experimental import pallas as pl
from jax.experimental.pallas import tpu as pltpu

_H = 112
_B = 32
_CT = 48
_EPS = 1e-08


def _build_cell_masks():
    g = np.zeros((16, _H, _H), dtype=np.float32)
    for i in range(16):
        hb, wb = divmod(i, 4)
        g[i, 28 * hb:28 * (hb + 1), 28 * wb:28 * (wb + 1)] = 1.0
    return g


_CELLS = _build_cell_masks()
_PERM = np.asarray(jax.random.permutation(jax.random.key(42), _B), dtype=np.int32)


def _imdiff_body(x_ref, o_ref):
    x = x_ref[0]
    vals = []
    for t in range(15):
        d = (jnp.abs(x[t] - x[t + 1]) + jnp.abs(x[16 + t] - x[17 + t])) \
            + jnp.abs(x[32 + t] - x[33 + t])
        vals.append(d)
    while len(vals) > 1:
        nxt = [vals[i] + vals[i + 1] for i in range(0, len(vals) - 1, 2)]
        if len(vals) % 2:
            nxt.append(vals[-1])
        vals = nxt
    o_ref[0] = vals[0] * np.float32(1.0 / 15.0)


def _select_body(perm_ref, m_ref, a_ref, b_ref, o_ref):
    del perm_ref
    m = m_ref[0] > 0.5
    o_ref[0] = jnp.where(m[None], a_ref[0], b_ref[0])


def _gaussian_kernel1d(ksize, sigma):
    x = jnp.arange(ksize, dtype=jnp.float32) - (ksize // 2)
    g = jnp.exp(-(x ** 2) / (2.0 * sigma * sigma))
    return g / g.sum()


def _gauss_blur(img, ksize, sigma):
    k1 = _gaussian_kernel1d(ksize, sigma)
    k2 = jnp.outer(k1, k1)
    k2 = k2 / k2.sum()
    pad = ksize // 2
    x = jnp.pad(img, ((0, 0), (0, 0), (pad, pad), (pad, pad)), mode='reflect')
    kern = k2[None, None, :, :]
    return jax.lax.conv_general_dilated(x, kern, (1, 1), 'VALID',
                                        dimension_numbers=('NCHW', 'OIHW', 'NCHW'))


def _ni_batch(m):
    b, h, w = m.shape
    f = m.reshape(b, -1)
    f = f - f.min(axis=-1, keepdims=True)
    f = f / (f.max(axis=-1, keepdims=True) + _EPS)
    return f.reshape(b, h, w)


def kernel(video_clips):
    b, c, t, h, w = video_clips.shape
    x = video_clips.reshape(b, c * t, h, w)
    grid_cells = jnp.asarray(_CELLS)

    im_diff = pl.pallas_call(
        _imdiff_body,
        grid=(b,),
        in_specs=[pl.BlockSpec((1, _CT, _H, _H), lambda i: (i, 0, 0, 0))],
        out_specs=pl.BlockSpec((1, _H, _H), lambda i: (i, 0, 0)),
        out_shape=jax.ShapeDtypeStruct((b, _H, _H), jnp.float32),
        compiler_params=pltpu.CompilerParams(
            dimension_semantics=("arbitrary",),
        ),
    )(x)

    gsize = int(0.1 * _H) // 2 * 2 + 1
    mask = _gauss_blur(im_diff.reshape(-1, 1, h, w), gsize, gsize / 3.0)
    mask = _ni_batch(mask.reshape(-1, h, w))
    activation = mask.reshape(b, -1) @ grid_cells.reshape(16, -1).T
    _, fg_index = jax.lax.top_k(activation, 8)
    selmask = grid_cells[fg_index.reshape(-1)].reshape(b, 8, h, w).sum(axis=1)

    perm = jnp.asarray(_PERM)
    grid_spec = pltpu.PrefetchScalarGridSpec(
        num_scalar_prefetch=1,
        grid=(b,),
        in_specs=[
            pl.BlockSpec((1, _H, _H), lambda i, p: (i, 0, 0)),
            pl.BlockSpec((1, _CT, _H, _H), lambda i, p: (i, 0, 0, 0)),
            pl.BlockSpec((1, _CT, _H, _H), lambda i, p: (p[i], 0, 0, 0)),
        ],
        out_specs=pl.BlockSpec((1, _CT, _H, _H), lambda i, p: (i, 0, 0, 0)),
    )
    out = pl.pallas_call(
        _select_body,
        grid_spec=grid_spec,
        out_shape=jax.ShapeDtypeStruct((b, c * t, h, w), jnp.float32),
        compiler_params=pltpu.CompilerParams(
            dimension_semantics=("arbitrary",),
        ),
    )(perm, selmask, x, x)
    return out.reshape(b, c, t, h, w)
